# baseline (speedup 1.0000x reference)
import jax
import jax.numpy as jnp
from jax.experimental import pallas as pl


TN = 1024
TM = 1024


def _chamfer_kernel(x1_ref, x2_ref, asq_ref, csq_ref, d1_ref, d2_ref):
    b = pl.program_id(0)
    i = pl.program_id(1)
    a = x1_ref[0]
    a_sq = asq_ref[b, pl.ds(i * TN, TN)]
    m = csq_ref.shape[1]
    rmin = None
    for mc in range(m // TM):
        c = x2_ref[0, pl.ds(mc * TM, TM), :]
        c_sq = csq_ref[b, pl.ds(mc * TM, TM)]
        nc = jax.lax.dot_general(
            a, c, (((1,), (1,)), ((), ())),
            preferred_element_type=jnp.float32)
        e = nc + c_sq[None, :]
        pmin = jnp.min(e, axis=1)
        rmin = pmin if rmin is None else jnp.minimum(rmin, pmin)
        part2 = jnp.min(e + a_sq[:, None], axis=0)[None, :]
        sl = (pl.ds(b, 1), pl.ds(mc * TM, TM))

        @pl.when(i == 0)
        def _():
            d2_ref[sl] = part2

        @pl.when(i != 0)
        def _():
            d2_ref[sl] = jnp.minimum(d2_ref[sl], part2)

    d1_ref[pl.ds(b, 1), pl.ds(i * TN, TN)] = jnp.maximum(
        rmin + a_sq, 0.0)[None, :]


@jax.jit
def kernel(xyz1, xyz2):
    B, N, _ = xyz1.shape
    M = xyz2.shape[1]
    a_sq = jnp.sum(xyz1 * xyz1, axis=2)
    b_sq = jnp.sum(xyz2 * xyz2, axis=2)
    grid = (B, N // TN)
    d1, d2 = pl.pallas_call(
        _chamfer_kernel,
        grid=grid,
        in_specs=[
            pl.BlockSpec((1, TN, 3), lambda b, i: (b, i, 0)),
            pl.BlockSpec((1, M, 3), lambda b, i: (b, 0, 0)),
            pl.BlockSpec((B, N), lambda b, i: (0, 0)),
            pl.BlockSpec((B, M), lambda b, i: (0, 0)),
        ],
        out_specs=[
            pl.BlockSpec((B, N), lambda b, i: (0, 0)),
            pl.BlockSpec((B, M), lambda b, i: (0, 0)),
        ],
        out_shape=[
            jax.ShapeDtypeStruct((B, N), jnp.float32),
            jax.ShapeDtypeStruct((B, M), jnp.float32),
        ],
    )(xyz1, -2.0 * xyz2, a_sq, b_sq)
    d2 = jnp.maximum(d2, 0.0)
    return (d1, d2)


# speedup vs baseline: 1.0239x; 1.0239x over previous
import jax
import jax.numpy as jnp
from jax.experimental import pallas as pl


TN = 1024


def _chamfer_kernel(x1_ref, x2_ref, asq_ref, csq_ref, d1_ref, d2_ref):
    i = pl.program_id(0)
    b = pl.program_id(1)
    a = x1_ref[0]
    c = x2_ref[0]
    a_sq = asq_ref[b, pl.ds(i * TN, TN)]
    c_sq = csq_ref[b, :]
    nc = jax.lax.dot_general(
        a, c, (((1,), (1,)), ((), ())),
        preferred_element_type=jnp.float32)
    e = nc + c_sq[None, :]
    d1_ref[pl.ds(b, 1), pl.ds(i * TN, TN)] = jnp.maximum(
        jnp.min(e, axis=1) + a_sq, 0.0)[None, :]
    part2 = jnp.min(e + a_sq[:, None], axis=0)[None, :]

    @pl.when(i == 0)
    def _():
        d2_ref[pl.ds(b, 1), :] = part2

    @pl.when(i != 0)
    def _():
        d2_ref[pl.ds(b, 1), :] = jnp.minimum(d2_ref[pl.ds(b, 1), :], part2)


@jax.jit
def kernel(xyz1, xyz2):
    B, N, _ = xyz1.shape
    M = xyz2.shape[1]
    a_sq = jnp.sum(xyz1 * xyz1, axis=2)
    b_sq = jnp.sum(xyz2 * xyz2, axis=2)
    grid = (N // TN, B)
    d1, d2 = pl.pallas_call(
        _chamfer_kernel,
        grid=grid,
        in_specs=[
            pl.BlockSpec((1, TN, 3), lambda i, b: (b, i, 0)),
            pl.BlockSpec((1, M, 3), lambda i, b: (b, 0, 0)),
            pl.BlockSpec((B, N), lambda i, b: (0, 0)),
            pl.BlockSpec((B, M), lambda i, b: (0, 0)),
        ],
        out_specs=[
            pl.BlockSpec((B, N), lambda i, b: (0, 0)),
            pl.BlockSpec((B, M), lambda i, b: (0, 0)),
        ],
        out_shape=[
            jax.ShapeDtypeStruct((B, N), jnp.float32),
            jax.ShapeDtypeStruct((B, M), jnp.float32),
        ],
    )(xyz1, -2.0 * xyz2, a_sq, b_sq)
    d2 = jnp.maximum(d2, 0.0)
    return (d1, d2)


# speedup vs baseline: 1.0260x; 1.0020x over previous
import jax
import jax.numpy as jnp
from jax.experimental import pallas as pl


TN = 1024


def _chamfer_kernel(x1_ref, x2_ref, asq_ref, csq_ref, d1_ref, d2_ref):
    b = pl.program_id(0)
    i = pl.program_id(1)
    a = x1_ref[0]
    c = x2_ref[0]
    a_sq = asq_ref[b, pl.ds(i * TN, TN)]
    c_sq = csq_ref[b, :]
    nc = jax.lax.dot_general(
        a, c, (((1,), (1,)), ((), ())),
        preferred_element_type=jnp.float32)
    e = nc + c_sq[None, :]
    d1_ref[pl.ds(b, 1), pl.ds(i * TN, TN)] = jnp.maximum(
        jnp.min(e, axis=1) + a_sq, 0.0)[None, :]
    part2 = jnp.min(e + a_sq[:, None], axis=0)[None, :]

    @pl.when(i == 0)
    def _():
        d2_ref[pl.ds(b, 1), :] = part2

    @pl.when(i != 0)
    def _():
        d2_ref[pl.ds(b, 1), :] = jnp.minimum(d2_ref[pl.ds(b, 1), :], part2)


@jax.jit
def kernel(xyz1, xyz2):
    B, N, _ = xyz1.shape
    M = xyz2.shape[1]
    a_sq = jnp.sum(xyz1 * xyz1, axis=2)
    b_sq = jnp.sum(xyz2 * xyz2, axis=2)
    grid = (B, N // TN)
    d1, d2 = pl.pallas_call(
        _chamfer_kernel,
        grid=grid,
        in_specs=[
            pl.BlockSpec((1, TN, 3), lambda b, i: (b, i, 0)),
            pl.BlockSpec((1, M, 3), lambda b, i: (b, 0, 0)),
            pl.BlockSpec((B, N), lambda b, i: (0, 0)),
            pl.BlockSpec((B, M), lambda b, i: (0, 0)),
        ],
        out_specs=[
            pl.BlockSpec((B, N), lambda b, i: (0, 0)),
            pl.BlockSpec((B, M), lambda b, i: (0, 0)),
        ],
        out_shape=[
            jax.ShapeDtypeStruct((B, N), jnp.float32),
            jax.ShapeDtypeStruct((B, M), jnp.float32),
        ],
    )(xyz1, -2.0 * xyz2, a_sq, b_sq)
    d2 = jnp.maximum(d2, 0.0)
    return (d1, d2)


# speedup vs baseline: 1.0522x; 1.0255x over previous
import jax
import jax.numpy as jnp
from jax.experimental import pallas as pl


TN = 512


def _chamfer_kernel(x1_ref, x2_ref, asq_ref, csq_ref, d1_ref, d2_ref):
    b = pl.program_id(0)
    i = pl.program_id(1)
    a = x1_ref[0]
    c = x2_ref[0]
    a_sq = asq_ref[b, pl.ds(i * TN, TN)]
    c_sq = csq_ref[b, :]
    nc = jax.lax.dot_general(
        a, c, (((1,), (1,)), ((), ())),
        preferred_element_type=jnp.float32)
    e = nc + c_sq[None, :]
    d1_ref[pl.ds(b, 1), pl.ds(i * TN, TN)] = jnp.maximum(
        jnp.min(e, axis=1) + a_sq, 0.0)[None, :]
    part2 = jnp.min(e + a_sq[:, None], axis=0)[None, :]

    @pl.when(i == 0)
    def _():
        d2_ref[pl.ds(b, 1), :] = part2

    @pl.when(i != 0)
    def _():
        d2_ref[pl.ds(b, 1), :] = jnp.minimum(d2_ref[pl.ds(b, 1), :], part2)


@jax.jit
def kernel(xyz1, xyz2):
    B, N, _ = xyz1.shape
    M = xyz2.shape[1]
    a_sq = jnp.sum(xyz1 * xyz1, axis=2)
    b_sq = jnp.sum(xyz2 * xyz2, axis=2)
    grid = (B, N // TN)
    d1, d2 = pl.pallas_call(
        _chamfer_kernel,
        grid=grid,
        in_specs=[
            pl.BlockSpec((1, TN, 3), lambda b, i: (b, i, 0)),
            pl.BlockSpec((1, M, 3), lambda b, i: (b, 0, 0)),
            pl.BlockSpec((B, N), lambda b, i: (0, 0)),
            pl.BlockSpec((B, M), lambda b, i: (0, 0)),
        ],
        out_specs=[
            pl.BlockSpec((B, N), lambda b, i: (0, 0)),
            pl.BlockSpec((B, M), lambda b, i: (0, 0)),
        ],
        out_shape=[
            jax.ShapeDtypeStruct((B, N), jnp.float32),
            jax.ShapeDtypeStruct((B, M), jnp.float32),
        ],
    )(xyz1, -2.0 * xyz2, a_sq, b_sq)
    d2 = jnp.maximum(d2, 0.0)
    return (d1, d2)
